# baseline (speedup 1.0000x reference)
import jax
import jax.numpy as jnp
from jax import lax
from jax.experimental import pallas as pl
from jax.experimental.pallas import tpu as pltpu
from jax.experimental.pallas import tpu_sc as plsc

B = 4096
D = 1024
K = 64
BB = 512

NC = 2
NS = 16
NW = NC * NS
R = 2 * B
RPW = R // NW
CH = 32
NCH = RPW // CH


def _proj_kernel(hf_ref, hs_ref, idx_ref, wt_ref, w_ref, wm_ref, c_ref, ws_ref):
    i = pl.program_id(0)

    @pl.when(i == 0)
    def _init():
        ws_ref[...] = jnp.zeros_like(ws_ref)

    dn = (((1,), (1,)), ((), ()))
    m = lax.dot_general(hf_ref[...], wm_ref[:, :D], dn,
                        preferred_element_type=jnp.float32)
    m += lax.dot_general(hs_ref[...], wm_ref[:, D:], dn,
                         preferred_element_type=jnp.float32)
    c_ref[0] = w_ref[:, 0:1] * m
    c_ref[1] = w_ref[:, 1:2] * m

    kidx = lax.broadcasted_iota(jnp.int32, (K, BB), 0)
    s_t = (jnp.where(kidx == idx_ref[0:1, :], wt_ref[0:1, :], 0.0)
           + jnp.where(kidx == idx_ref[1:2, :], wt_ref[1:2, :], 0.0))
    ws_ref[...] += jnp.broadcast_to(
        jnp.sum(s_t, axis=1, keepdims=True), (K, 128))


def _sc_scatter(c_hbm, idx_hbm, out_hbm, acc, idx_sh, idx_s, in_buf):
    cid = lax.axis_index("c")
    sid = lax.axis_index("s")
    wid = sid * NC + cid

    zero = jnp.zeros((16,), jnp.float32)

    def zrow_body(r, carry):
        for j in range(D // 16):
            acc[r, pl.ds(j * 16, 16)] = zero
        return carry

    lax.fori_loop(0, K, zrow_body, 0)

    pltpu.sync_copy(idx_hbm.at[pl.ds(wid * NCH, NCH)], idx_sh.at[sid])
    pltpu.sync_copy(idx_sh.at[sid], idx_s)

    def chunk_body(c, carry):
        base = wid * RPW + c * CH
        pltpu.sync_copy(c_hbm.at[pl.ds(base, CH)], in_buf)

        def row_body(t, carry2):
            r = idx_s[c, t]
            for j in range(D // 16):
                sl = pl.ds(j * 16, 16)
                acc[r, sl] = acc[r, sl] + in_buf[t, sl]
            return carry2

        lax.fori_loop(0, CH, row_body, 0)
        return carry

    lax.fori_loop(0, NCH, chunk_body, 0)

    pltpu.sync_copy(acc, out_hbm.at[wid])


_sc_scatter_fn = pl.kernel(
    _sc_scatter,
    out_type=jax.ShapeDtypeStruct((NW, K, D), jnp.float32),
    mesh=plsc.VectorSubcoreMesh(core_axis_name="c", subcore_axis_name="s"),
    scratch_types=[
        pltpu.VMEM((K, D), jnp.float32),
        pltpu.VMEM_SHARED((NS, NCH, CH), jnp.int32),
        pltpu.SMEM((NCH, CH), jnp.int32),
        pltpu.VMEM((CH, D), jnp.float32),
    ],
)


def _gru_kernel(agg_ref, ws_ref, a_ref, wih_ref, whh_ref, bih_ref, bhh_ref,
                out_ref, am_scr, r_scr, z_scr):
    j = pl.program_id(0)
    dn = (((1,), (1,)), ((), ()))

    @pl.when(j == 0)
    def _mean():
        ws = ws_ref[:, 0:1]
        am_scr[...] = jnp.sum(agg_ref[...], axis=0) / (ws + 1e-9)

    am = am_scr[...]
    a = a_ref[...]
    bih = bih_ref[0]
    bhh = bhh_ref[0]
    gi = lax.dot_general(am, wih_ref[...], dn, preferred_element_type=jnp.float32)
    gh = lax.dot_general(a, whh_ref[...], dn, preferred_element_type=jnp.float32)

    @pl.when(j == 0)
    def _r():
        r_scr[...] = jax.nn.sigmoid(gi + gh + bih + bhh)

    @pl.when(j == 1)
    def _z():
        z_scr[...] = jax.nn.sigmoid(gi + gh + bih + bhh)

    @pl.when(j == 2)
    def _n():
        i_n = gi + bih
        h_n = gh + bhh
        n = jnp.tanh(i_n + r_scr[...] * h_n)
        z = z_scr[...]
        new = (1.0 - z) * n + z * a
        used = ws_ref[:, 0:1] > 0.0
        out_ref[...] = jnp.where(used, new, a)


@jax.jit
def kernel(h_fast, h_slow, idx, weight, A_states, W_m, W_ih, W_hh, b_ih, b_hh):
    idx32 = idx.astype(jnp.int32)
    idx_t = idx32.T
    w_t = weight.T

    grid = B // BB
    c, wsum = pl.pallas_call(
        _proj_kernel,
        grid=(grid,),
        in_specs=[
            pl.BlockSpec((BB, D), lambda i: (i, 0)),
            pl.BlockSpec((BB, D), lambda i: (i, 0)),
            pl.BlockSpec((2, BB), lambda i: (0, i)),
            pl.BlockSpec((2, BB), lambda i: (0, i)),
            pl.BlockSpec((BB, 2), lambda i: (i, 0)),
            pl.BlockSpec((D, 2 * D), lambda i: (0, 0)),
        ],
        out_specs=[
            pl.BlockSpec((2, BB, D), lambda i: (0, i, 0)),
            pl.BlockSpec((K, 128), lambda i: (0, 0)),
        ],
        out_shape=[
            jax.ShapeDtypeStruct((2, B, D), jnp.float32),
            jax.ShapeDtypeStruct((K, 128), jnp.float32),
        ],
        compiler_params=pltpu.CompilerParams(
            dimension_semantics=("arbitrary",),
        ),
    )(h_fast, h_slow, idx_t, w_t, weight, W_m)

    c_flat = c.reshape(R, D)
    idx_flat = idx_t.reshape(R // CH, CH)
    partials = _sc_scatter_fn(c_flat, idx_flat)

    bih2 = b_ih.reshape(3, 1, D)
    bhh2 = b_hh.reshape(3, 1, D)
    updated = pl.pallas_call(
        _gru_kernel,
        grid=(3,),
        in_specs=[
            pl.BlockSpec((NW, K, D), lambda j: (0, 0, 0)),
            pl.BlockSpec((K, 128), lambda j: (0, 0)),
            pl.BlockSpec((K, D), lambda j: (0, 0)),
            pl.BlockSpec((D, D), lambda j: (j, 0)),
            pl.BlockSpec((D, D), lambda j: (j, 0)),
            pl.BlockSpec((1, 1, D), lambda j: (j, 0, 0)),
            pl.BlockSpec((1, 1, D), lambda j: (j, 0, 0)),
        ],
        out_specs=pl.BlockSpec((K, D), lambda j: (0, 0)),
        out_shape=jax.ShapeDtypeStruct((K, D), jnp.float32),
        scratch_shapes=[
            pltpu.VMEM((K, D), jnp.float32),
            pltpu.VMEM((K, D), jnp.float32),
            pltpu.VMEM((K, D), jnp.float32),
        ],
        compiler_params=pltpu.CompilerParams(
            dimension_semantics=("arbitrary",),
        ),
    )(partials, wsum, A_states, W_ih, W_hh, bih2, bhh2)
    return updated


# speedup vs baseline: 1.1352x; 1.1352x over previous
import jax
import jax.numpy as jnp
from jax import lax
from jax.experimental import pallas as pl
from jax.experimental.pallas import tpu as pltpu
from jax.experimental.pallas import tpu_sc as plsc

B = 4096
D = 1024
K = 64
BB = 512

NC = 2
NS = 16
NW = NC * NS
R = 2 * B
RPW = R // NW
CH = 32
NCH = RPW // CH


def _proj_kernel(hf_ref, hs_ref, idx_ref, wt_ref, w_ref, wm_ref, c_ref, ws_ref):
    i = pl.program_id(0)

    @pl.when(i == 0)
    def _init():
        ws_ref[...] = jnp.zeros_like(ws_ref)

    dn = (((1,), (1,)), ((), ()))
    m = lax.dot_general(hf_ref[...], wm_ref[:, :D], dn,
                        preferred_element_type=jnp.float32)
    m += lax.dot_general(hs_ref[...], wm_ref[:, D:], dn,
                         preferred_element_type=jnp.float32)
    c_ref[0] = w_ref[:, 0:1] * m
    c_ref[1] = w_ref[:, 1:2] * m

    kidx = lax.broadcasted_iota(jnp.int32, (K, BB), 0)
    s_t = (jnp.where(kidx == idx_ref[0:1, :], wt_ref[0:1, :], 0.0)
           + jnp.where(kidx == idx_ref[1:2, :], wt_ref[1:2, :], 0.0))
    ws_ref[...] += jnp.broadcast_to(
        jnp.sum(s_t, axis=1, keepdims=True), (K, 128))


def _sc_scatter(c_hbm, idx_hbm, out_hbm, acc, idx_sh, idx_s, in_buf):
    cid = lax.axis_index("c")
    sid = lax.axis_index("s")
    wid = sid * NC + cid

    zero = jnp.zeros((16,), jnp.float32)

    def zrow_body(r, carry):
        for j in range(D // 16):
            acc[r, pl.ds(j * 16, 16)] = zero
        return carry

    lax.fori_loop(0, K, zrow_body, 0)

    pltpu.sync_copy(idx_hbm.at[pl.ds(wid * NCH, NCH)], idx_sh.at[sid])
    pltpu.sync_copy(idx_sh.at[sid], idx_s)

    def chunk_body(c, carry):
        base = wid * RPW + c * CH
        pltpu.sync_copy(c_hbm.at[pl.ds(base, CH)], in_buf)

        def row_body(t, carry2):
            r = idx_s[c, t]
            for j in range(D // 16):
                sl = pl.ds(j * 16, 16)
                plsc.addupdate(acc.at[r, sl], in_buf[t, sl])
            return carry2

        lax.fori_loop(0, CH, row_body, 0)
        return carry

    lax.fori_loop(0, NCH, chunk_body, 0)

    pltpu.sync_copy(acc, out_hbm.at[wid])


_sc_scatter_fn = pl.kernel(
    _sc_scatter,
    out_type=jax.ShapeDtypeStruct((NW, K, D), jnp.float32),
    mesh=plsc.VectorSubcoreMesh(core_axis_name="c", subcore_axis_name="s"),
    scratch_types=[
        pltpu.VMEM((K, D), jnp.float32),
        pltpu.VMEM_SHARED((NS, NCH, CH), jnp.int32),
        pltpu.SMEM((NCH, CH), jnp.int32),
        pltpu.VMEM((CH, D), jnp.float32),
    ],
)


def _gru_kernel(agg_ref, ws_ref, a_ref, wih_ref, whh_ref, bih_ref, bhh_ref,
                out_ref, am_scr, r_scr, z_scr):
    j = pl.program_id(0)
    dn = (((1,), (1,)), ((), ()))

    @pl.when(j == 0)
    def _mean():
        ws = ws_ref[:, 0:1]
        am_scr[...] = jnp.sum(agg_ref[...], axis=0) / (ws + 1e-9)

    am = am_scr[...]
    a = a_ref[...]
    bih = bih_ref[0]
    bhh = bhh_ref[0]
    gi = lax.dot_general(am, wih_ref[...], dn, preferred_element_type=jnp.float32)
    gh = lax.dot_general(a, whh_ref[...], dn, preferred_element_type=jnp.float32)

    @pl.when(j == 0)
    def _r():
        r_scr[...] = jax.nn.sigmoid(gi + gh + bih + bhh)

    @pl.when(j == 1)
    def _z():
        z_scr[...] = jax.nn.sigmoid(gi + gh + bih + bhh)

    @pl.when(j == 2)
    def _n():
        i_n = gi + bih
        h_n = gh + bhh
        n = jnp.tanh(i_n + r_scr[...] * h_n)
        z = z_scr[...]
        new = (1.0 - z) * n + z * a
        used = ws_ref[:, 0:1] > 0.0
        out_ref[...] = jnp.where(used, new, a)


@jax.jit
def kernel(h_fast, h_slow, idx, weight, A_states, W_m, W_ih, W_hh, b_ih, b_hh):
    idx32 = idx.astype(jnp.int32)
    idx_t = idx32.T
    w_t = weight.T

    grid = B // BB
    c, wsum = pl.pallas_call(
        _proj_kernel,
        grid=(grid,),
        in_specs=[
            pl.BlockSpec((BB, D), lambda i: (i, 0)),
            pl.BlockSpec((BB, D), lambda i: (i, 0)),
            pl.BlockSpec((2, BB), lambda i: (0, i)),
            pl.BlockSpec((2, BB), lambda i: (0, i)),
            pl.BlockSpec((BB, 2), lambda i: (i, 0)),
            pl.BlockSpec((D, 2 * D), lambda i: (0, 0)),
        ],
        out_specs=[
            pl.BlockSpec((2, BB, D), lambda i: (0, i, 0)),
            pl.BlockSpec((K, 128), lambda i: (0, 0)),
        ],
        out_shape=[
            jax.ShapeDtypeStruct((2, B, D), jnp.float32),
            jax.ShapeDtypeStruct((K, 128), jnp.float32),
        ],
        compiler_params=pltpu.CompilerParams(
            dimension_semantics=("arbitrary",),
        ),
    )(h_fast, h_slow, idx_t, w_t, weight, W_m)

    c_flat = c.reshape(R, D)
    idx_flat = idx_t.reshape(R // CH, CH)
    partials = _sc_scatter_fn(c_flat, idx_flat)

    bih2 = b_ih.reshape(3, 1, D)
    bhh2 = b_hh.reshape(3, 1, D)
    updated = pl.pallas_call(
        _gru_kernel,
        grid=(3,),
        in_specs=[
            pl.BlockSpec((NW, K, D), lambda j: (0, 0, 0)),
            pl.BlockSpec((K, 128), lambda j: (0, 0)),
            pl.BlockSpec((K, D), lambda j: (0, 0)),
            pl.BlockSpec((D, D), lambda j: (j, 0)),
            pl.BlockSpec((D, D), lambda j: (j, 0)),
            pl.BlockSpec((1, 1, D), lambda j: (j, 0, 0)),
            pl.BlockSpec((1, 1, D), lambda j: (j, 0, 0)),
        ],
        out_specs=pl.BlockSpec((K, D), lambda j: (0, 0)),
        out_shape=jax.ShapeDtypeStruct((K, D), jnp.float32),
        scratch_shapes=[
            pltpu.VMEM((K, D), jnp.float32),
            pltpu.VMEM((K, D), jnp.float32),
            pltpu.VMEM((K, D), jnp.float32),
        ],
        compiler_params=pltpu.CompilerParams(
            dimension_semantics=("arbitrary",),
        ),
    )(partials, wsum, A_states, W_ih, W_hh, bih2, bhh2)
    return updated
